# Initial kernel scaffold; baseline (speedup 1.0000x reference)
import jax
import jax.numpy as jnp
from jax.experimental import pallas as pl


def kernel(x, edge_index, batch, W1, b1, p1, W2, b2, p2, W3, b3, p3, lW1, lb1, lW2, lb2, lW3, lb3):
    raise NotImplementedError("write your pallas kernel here")


# speedup vs baseline: 1.0636x; 1.0636x over previous
import functools
import jax
import jax.numpy as jnp
from jax.experimental import pallas as pl
from jax.experimental.pallas import tpu as pltpu

N_NODES = 10000
N_EDGES = 320000
NFEAT = 128
NHID = 128
NCLS = 10
NGRAPH = 16
RATIO = 0.5


def _matmul_body(x_ref, w_ref, b_ref, o_ref):
    o_ref[...] = jnp.dot(x_ref[...], w_ref[...],
                         preferred_element_type=jnp.float32) + b_ref[...]


def _pallas_linear(x, W, b):
    M, K = x.shape
    Kw, Nout = W.shape
    blk = 1024 if M % 1024 == 0 else M
    grid = (M // blk,)
    return pl.pallas_call(
        _matmul_body,
        grid=grid,
        in_specs=[
            pl.BlockSpec((blk, K), lambda i: (i, 0)),
            pl.BlockSpec((K, Nout), lambda i: (0, 0)),
            pl.BlockSpec((Nout,), lambda i: (0,)),
        ],
        out_specs=pl.BlockSpec((blk, Nout), lambda i: (i, 0)),
        out_shape=jax.ShapeDtypeStruct((M, Nout), jnp.float32),
    )(x, W, b)


def _gcn_conv(x, src, dst, emask, W, b, p, N):
    h = _pallas_linear(x, W, b)
    deg = jax.ops.segment_sum(emask, dst, num_segments=N) + 1.0
    norm = emask / jnp.sqrt(deg[src] * deg[dst])
    agg = jax.ops.segment_sum(h[src] * norm[:, None], dst, num_segments=N) \
        + h / deg[:, None]
    score = jnp.tanh(agg @ p)
    return agg, score


def _att_pool(h, score, src, dst, emask, batch, ratio):
    N = h.shape[0]
    k = int(N * ratio)
    vals, perm = jax.lax.top_k(score, k)
    xn = h[perm] * jnp.tanh(vals)[:, None]
    mapping = jnp.full((N,), -1, dtype=jnp.int32).at[perm].set(
        jnp.arange(k, dtype=jnp.int32))
    ns = mapping[src]
    nd = mapping[dst]
    valid = emask * (ns >= 0).astype(h.dtype) * (nd >= 0).astype(h.dtype)
    return xn, jnp.maximum(ns, 0), jnp.maximum(nd, 0), valid, batch[perm]


def _readout(h, batch, B):
    cnt = jax.ops.segment_sum(jnp.ones((h.shape[0],), h.dtype), batch,
                              num_segments=B)
    s = jax.ops.segment_sum(h, batch, num_segments=B)
    mean = s / jnp.maximum(cnt, 1.0)[:, None]
    mx = jax.ops.segment_max(h, batch, num_segments=B)
    mx = jnp.where(cnt[:, None] > 0, mx, 0.0)
    return jnp.concatenate([mx, mean], axis=1)


def kernel(x, edge_index, batch, W1, b1, p1, W2, b2, p2, W3, b3, p3,
           lW1, lb1, lW2, lb2, lW3, lb3):
    src = edge_index[0].astype(jnp.int32)
    dst = edge_index[1].astype(jnp.int32)
    batch = batch.astype(jnp.int32)
    emask = jnp.ones((src.shape[0],), jnp.float32)

    h, s = _gcn_conv(x, src, dst, emask, W1, b1, p1, x.shape[0])
    h = jax.nn.relu(h)
    h, src, dst, emask, batch = _att_pool(h, s, src, dst, emask, batch, RATIO)
    x1 = _readout(h, batch, NGRAPH)

    h2, s = _gcn_conv(h, src, dst, emask, W2, b2, p2, h.shape[0])
    h2 = jax.nn.relu(h2)
    h2, src, dst, emask, batch = _att_pool(h2, s, src, dst, emask, batch, RATIO)
    x2 = _readout(h2, batch, NGRAPH)

    h3, s = _gcn_conv(h2, src, dst, emask, W3, b3, p3, h2.shape[0])
    h3 = jax.nn.relu(h3)
    x3 = _readout(h3, batch, NGRAPH)

    z = jax.nn.relu(x1) + jax.nn.relu(x2) + jax.nn.relu(x3)
    z = jax.nn.relu(z @ lW1 + lb1)
    z = jax.nn.relu(z @ lW2 + lb2)
    out = jax.nn.log_softmax(z @ lW3 + lb3, axis=-1)
    return out


# speedup vs baseline: 15.1838x; 14.2753x over previous
import dataclasses
import functools
import jax
import jax.numpy as jnp
from jax import lax
from jax.experimental import pallas as pl
from jax.experimental.pallas import tpu as pltpu
from jax.experimental.pallas import tpu_sc as plsc

N_NODES = 10000
NFEAT = 128
NHID = 128
NCLS = 10
NGRAPH = 16

N_SC = 2
N_SUB = 16
N_TILES = N_SC * N_SUB
CHUNK = 128
CHUNKS_PER_TILE = 79
EPT = CHUNK * CHUNKS_PER_TILE
E_PAD = EPT * N_TILES
N_ACC = 10240
DUMMY = N_NODES
ROWS_PER_TILE = N_ACC // N_SUB

_SC_MESH = plsc.VectorSubcoreMesh(
    core_axis_name="c", subcore_axis_name="s",
    num_cores=N_SC, num_subcores=N_SUB)

_SC_PARAMS = pltpu.CompilerParams()
if "needs_layout_passes" in pltpu.CompilerParams.__dataclass_fields__:
    _SC_PARAMS = dataclasses.replace(_SC_PARAMS, needs_layout_passes=False)


def _mask_deg_body(src_hbm, dst_hbm, alive_hbm, deg_out, dsteff_out,
                   src_ts, dst_ts, alive_ts, ones_ts, dsteff_ts, zrow_ts,
                   deg_spmem):
    c = lax.axis_index("c")
    s = lax.axis_index("s")
    t = c * N_SUB + s
    pltpu.sync_copy(alive_hbm, alive_ts)
    pltpu.sync_copy(src_hbm.at[t], src_ts)
    pltpu.sync_copy(dst_hbm.at[t], dst_ts)

    zeros16 = jnp.zeros((16,), jnp.float32)
    ones16 = jnp.ones((16,), jnp.float32)

    @pl.loop(0, ROWS_PER_TILE // 16)
    def _(i):
        zrow_ts[pl.ds(i * 16, 16)] = zeros16

    @pl.loop(0, CHUNK // 16)
    def _(i):
        ones_ts[0, pl.ds(i * 16, 16)] = ones16

    pltpu.sync_copy(zrow_ts, deg_spmem.at[pl.ds(s * ROWS_PER_TILE,
                                                ROWS_PER_TILE)])
    plsc.subcore_barrier()

    @pl.loop(0, CHUNKS_PER_TILE)
    def _(j):
        @pl.loop(0, CHUNK // 16)
        def _(g):
            si = src_ts[j, pl.ds(g * 16, 16)]
            di = dst_ts[j, pl.ds(g * 16, 16)]
            av = (plsc.load_gather(alive_ts, [si]) *
                  plsc.load_gather(alive_ts, [di]))
            dsteff_ts[j, pl.ds(g * 16, 16)] = jnp.where(
                av > 0, di, jnp.int32(DUMMY))
        pltpu.sync_copy(ones_ts.at[0], deg_spmem.at[dsteff_ts.at[j]],
                        add=True)

    plsc.subcore_barrier()
    pltpu.sync_copy(deg_spmem.at[pl.ds(s * ROWS_PER_TILE, ROWS_PER_TILE)],
                    deg_out.at[c, pl.ds(s * ROWS_PER_TILE, ROWS_PER_TILE)])
    pltpu.sync_copy(dsteff_ts, dsteff_out.at[t])


_sc_mask_deg = pl.kernel(
    _mask_deg_body,
    out_type=[
        jax.ShapeDtypeStruct((N_SC, N_ACC), jnp.float32),
        jax.ShapeDtypeStruct((N_TILES, CHUNKS_PER_TILE, CHUNK), jnp.int32),
    ],
    mesh=_SC_MESH,
    scratch_types=[
        pltpu.VMEM((CHUNKS_PER_TILE, CHUNK), jnp.int32),
        pltpu.VMEM((CHUNKS_PER_TILE, CHUNK), jnp.int32),
        pltpu.VMEM((N_ACC,), jnp.int32),
        pltpu.VMEM((1, CHUNK), jnp.float32),
        pltpu.VMEM((CHUNKS_PER_TILE, CHUNK), jnp.int32),
        pltpu.VMEM((ROWS_PER_TILE,), jnp.float32),
        pltpu.VMEM_SHARED((N_ACC,), jnp.float32),
    ],
    compiler_params=_SC_PARAMS,
)


def _aggregate_body(hp_hbm, src_hbm, dsteff_hbm, s_out,
                    src_ts, dsteff_ts, rows_ts, acc_spmem):
    c = lax.axis_index("c")
    s = lax.axis_index("s")
    t = c * N_SUB + s
    pltpu.sync_copy(src_hbm.at[t], src_ts)
    pltpu.sync_copy(dsteff_hbm.at[t], dsteff_ts)

    zeros16 = jnp.zeros((16,), jnp.float32)

    @pl.loop(0, CHUNK)
    def _(i):
        @pl.loop(0, CHUNK // 16)
        def _(g):
            rows_ts[i, pl.ds(g * 16, 16)] = zeros16

    @pl.loop(0, ROWS_PER_TILE // CHUNK)
    def _(i):
        pltpu.sync_copy(
            rows_ts, acc_spmem.at[pl.ds(s * ROWS_PER_TILE + i * CHUNK,
                                        CHUNK)])
    plsc.subcore_barrier()

    @pl.loop(0, CHUNKS_PER_TILE)
    def _(j):
        pltpu.sync_copy(hp_hbm.at[src_ts.at[j]], rows_ts)
        pltpu.sync_copy(rows_ts, acc_spmem.at[dsteff_ts.at[j]], add=True)

    plsc.subcore_barrier()

    @pl.loop(0, ROWS_PER_TILE // CHUNK)
    def _(i):
        off = s * ROWS_PER_TILE + i * CHUNK
        pltpu.sync_copy(acc_spmem.at[pl.ds(off, CHUNK)],
                        s_out.at[c, pl.ds(off, CHUNK)])


_sc_aggregate = pl.kernel(
    _aggregate_body,
    out_type=[jax.ShapeDtypeStruct((N_SC, N_ACC, NHID), jnp.float32)],
    mesh=_SC_MESH,
    scratch_types=[
        pltpu.VMEM((CHUNKS_PER_TILE, CHUNK), jnp.int32),
        pltpu.VMEM((CHUNKS_PER_TILE, CHUNK), jnp.int32),
        pltpu.VMEM((CHUNK, NHID), jnp.float32),
        pltpu.VMEM_SHARED((N_ACC, NHID), jnp.float32),
    ],
    compiler_params=_SC_PARAMS,
)


_BLK = 2000


def _linear_body(x_ref, w_ref, b_ref, o_ref):
    o_ref[...] = jnp.dot(x_ref[...], w_ref[...],
                         preferred_element_type=jnp.float32) + b_ref[...]


def _linear(x, W, b):
    M, K = x.shape
    _, Nout = W.shape
    return pl.pallas_call(
        _linear_body,
        grid=(M // _BLK,),
        in_specs=[
            pl.BlockSpec((_BLK, K), lambda i: (i, 0)),
            pl.BlockSpec((K, Nout), lambda i: (0, 0)),
            pl.BlockSpec((Nout,), lambda i: (0,)),
        ],
        out_specs=pl.BlockSpec((_BLK, Nout), lambda i: (i, 0)),
        out_shape=jax.ShapeDtypeStruct((M, Nout), jnp.float32),
    )(x, W, b)


def _prep_body(dp_ref, h_ref, hp_ref, r_ref):
    deg = dp_ref[:, 0] + dp_ref[:, 1] + 1.0
    r = lax.rsqrt(deg)
    hp_ref[...] = h_ref[...] * r[:, None]
    r_ref[...] = r[:, None]


def _prep(deg_partsT, h):
    return pl.pallas_call(
        _prep_body,
        grid=(N_NODES // _BLK,),
        in_specs=[
            pl.BlockSpec((_BLK, N_SC), lambda i: (i, 0)),
            pl.BlockSpec((_BLK, NHID), lambda i: (i, 0)),
        ],
        out_specs=[
            pl.BlockSpec((_BLK, NHID), lambda i: (i, 0)),
            pl.BlockSpec((_BLK, 1), lambda i: (i, 0)),
        ],
        out_shape=[
            jax.ShapeDtypeStruct((N_NODES, NHID), jnp.float32),
            jax.ShapeDtypeStruct((N_NODES, 1), jnp.float32),
        ],
    )(deg_partsT, h)


def _post_body(sp_ref, hp_ref, r_ref, p_ref, hrelu_ref, score_ref):
    ssum = sp_ref[0] + sp_ref[1]
    agg = (ssum + hp_ref[...]) * r_ref[...]
    score_ref[...] = jnp.tanh(
        jnp.dot(agg, p_ref[...], preferred_element_type=jnp.float32))
    hrelu_ref[...] = jnp.maximum(agg, 0.0)


def _post(s_parts, hp, r, p2):
    return pl.pallas_call(
        _post_body,
        grid=(N_NODES // _BLK,),
        in_specs=[
            pl.BlockSpec((N_SC, _BLK, NHID), lambda i: (0, i, 0)),
            pl.BlockSpec((_BLK, NHID), lambda i: (i, 0)),
            pl.BlockSpec((_BLK, 1), lambda i: (i, 0)),
            pl.BlockSpec((NHID, 1), lambda i: (0, 0)),
        ],
        out_specs=[
            pl.BlockSpec((_BLK, NHID), lambda i: (i, 0)),
            pl.BlockSpec((_BLK, 1), lambda i: (i, 0)),
        ],
        out_shape=[
            jax.ShapeDtypeStruct((N_NODES, NHID), jnp.float32),
            jax.ShapeDtypeStruct((N_NODES, 1), jnp.float32),
        ],
    )(s_parts, hp, r, p2)


def _pool_body(k, score_ref, alive_ref, h_ref, xn_ref, alive_out_ref):
    score = score_ref[...]
    alive = alive_ref[...]
    int_min = jnp.int32(-2**31)
    bits = lax.bitcast_convert_type(score, jnp.int32)
    skey = bits ^ ((bits >> 31) & jnp.int32(0x7FFFFFFF))
    skey = jnp.where(alive > 0, skey, int_min)
    kk = jnp.int32(k)

    cpos = jnp.sum((skey >= 0).astype(jnp.int32))
    lo = jnp.where(cpos >= kk, jnp.int32(0), int_min)
    hi = jnp.where(cpos >= kk, jnp.int32(2**31 - 1), jnp.int32(-1))

    def step(_, lh):
        lo, hi = lh
        mid = lo + ((hi - lo) >> 1)
        cgt = jnp.sum((skey > mid).astype(jnp.int32))
        take_hi = cgt < kk
        return (jnp.where(take_hi, lo, mid + 1),
                jnp.where(take_hi, mid, hi))

    lo, _ = lax.fori_loop(0, 31, step, (lo, hi))
    thr = lo
    need = kk - jnp.sum((skey > thr).astype(jnp.int32))
    ties = skey == thr
    idx = lax.broadcasted_iota(jnp.int32, score.shape, 0)

    def step2(_, lh):
        lo, hi = lh
        mid = (lo + hi) >> 1
        cnt = jnp.sum((ties & (idx < mid)).astype(jnp.int32))
        take_hi = cnt >= need
        return (jnp.where(take_hi, lo, mid + 1),
                jnp.where(take_hi, mid, hi))

    jstar, _ = lax.fori_loop(0, 14, step2,
                             (jnp.int32(0), jnp.int32(N_NODES)))
    sel = (skey > thr) | (ties & (idx < jstar))
    alive_out_ref[...] = sel.astype(jnp.float32)
    xn_ref[...] = jnp.where(sel, h_ref[...] * jnp.tanh(score), 0.0)


def _pool(score, alive, h, k):
    return pl.pallas_call(
        functools.partial(_pool_body, k),
        out_shape=[
            jax.ShapeDtypeStruct((N_NODES, NHID), jnp.float32),
            jax.ShapeDtypeStruct((N_NODES, 1), jnp.float32),
        ],
    )(score, alive, h)


def _readout_body(h_ref, batch_ref, alive_ref, out_ref):
    h = h_ref[...]
    b = batch_ref[...]
    alive = alive_ref[...] > 0
    gid = lax.broadcasted_iota(jnp.int32, (N_NODES, NGRAPH), 1)
    m = (b == gid) & alive
    mf = m.astype(jnp.float32)
    cnt = jnp.sum(mf, axis=0)
    sums = lax.dot_general(mf, h, (((0,), (0,)), ((), ())),
                           preferred_element_type=jnp.float32)
    mean = sums / jnp.maximum(cnt, 1.0)[:, None]
    neg = jnp.float32(-3.0e38)
    rowid = lax.broadcasted_iota(jnp.int32, (NGRAPH, 1), 0)

    def body(g, acc):
        mg = (b == g) & alive
        col = jnp.max(jnp.where(mg, h, neg), axis=0)
        return jnp.where(rowid == g, col[None, :], acc)

    mx = lax.fori_loop(0, NGRAPH, body,
                       jnp.zeros((NGRAPH, NHID), jnp.float32))
    mx = jnp.where(cnt[:, None] > 0, mx, 0.0)
    out_ref[:, 0:NHID] = mx
    out_ref[:, NHID:] = mean


def _readout(h, batch2, alive):
    return pl.pallas_call(
        _readout_body,
        out_shape=jax.ShapeDtypeStruct((NGRAPH, 2 * NHID), jnp.float32),
    )(h, batch2, alive)


def _mlp_body(x1_ref, x2_ref, x3_ref, w1_ref, b1_ref, w2_ref, b2_ref,
              w3_ref, b3_ref, out_ref):
    z = (jnp.maximum(x1_ref[...], 0.0) + jnp.maximum(x2_ref[...], 0.0) +
         jnp.maximum(x3_ref[...], 0.0))
    z = jnp.maximum(jnp.dot(z, w1_ref[...],
                            preferred_element_type=jnp.float32)
                    + b1_ref[...], 0.0)
    z = jnp.maximum(jnp.dot(z, w2_ref[...],
                            preferred_element_type=jnp.float32)
                    + b2_ref[...], 0.0)
    o = jnp.dot(z, w3_ref[...], preferred_element_type=jnp.float32) \
        + b3_ref[...]
    mx = jnp.max(o, axis=-1, keepdims=True)
    lse = jnp.log(jnp.sum(jnp.exp(o - mx), axis=-1, keepdims=True)) + mx
    out_ref[...] = o - lse


def _mlp(x1, x2, x3, lW1, lb1, lW2, lb2, lW3, lb3):
    return pl.pallas_call(
        _mlp_body,
        out_shape=jax.ShapeDtypeStruct((NGRAPH, NCLS), jnp.float32),
    )(x1, x2, x3, lW1, lb1, lW2, lb2, lW3, lb3)


def kernel(x, edge_index, batch, W1, b1, p1, W2, b2, p2, W3, b3, p3,
           lW1, lb1, lW2, lb2, lW3, lb3):
    src = edge_index[0].astype(jnp.int32)
    dst = edge_index[1].astype(jnp.int32)
    batch2 = batch.astype(jnp.int32)[:, None]
    n_edges = src.shape[0]
    pad = E_PAD - n_edges
    src3 = jnp.concatenate(
        [src, jnp.zeros((pad,), jnp.int32)]).reshape(
            N_TILES, CHUNKS_PER_TILE, CHUNK)
    dst3 = jnp.concatenate(
        [dst, jnp.full((pad,), DUMMY, jnp.int32)]).reshape(
            N_TILES, CHUNKS_PER_TILE, CHUNK)

    def layer(xin, alive_f, W, b, p):
        alive_i = jnp.concatenate(
            [(alive_f[:, 0] > 0).astype(jnp.int32),
             jnp.zeros((N_ACC - N_NODES,), jnp.int32)])
        h = _linear(xin, W, b)
        deg_parts, dsteff = _sc_mask_deg(src3, dst3, alive_i)
        hp, r = _prep(deg_parts.T, h)
        (s_parts,) = _sc_aggregate(hp, src3, dsteff)
        return _post(s_parts, hp, r, p[:, None])

    alive0 = jnp.ones((N_NODES, 1), jnp.float32)
    h1, score1 = layer(x, alive0, W1, b1, p1)
    xn1, alive1 = _pool(score1, alive0, h1, N_NODES // 2)
    x1 = _readout(xn1, batch2, alive1)

    h2, score2 = layer(xn1, alive1, W2, b2, p2)
    xn2, alive2 = _pool(score2, alive1, h2, N_NODES // 4)
    x2 = _readout(xn2, batch2, alive2)

    h3, _ = layer(xn2, alive2, W3, b3, p3)
    x3 = _readout(h3, batch2, alive2)

    return _mlp(x1, x2, x3, lW1, lb1, lW2, lb2, lW3, lb3)
